# baseline (speedup 1.0000x reference)
import jax
import jax.numpy as jnp
import numpy as np
from jax.experimental import pallas as pl
from jax.experimental.pallas import tpu as pltpu

B, L, C = 8, 512, 321
PRED, PATCH, K, CC, ALPHA = 96, 16, 3, 2, 0.5
E, TOPK, FF = 8, 2, 2048
NBLOCK = 2
EPS = 1e-5

CP = 384
NTOK = B * CP
BT = 1024
NT = NTOK // BT
PP = 128
NPATCH = L // PATCH


def _sel_mats(n):
    ev = np.zeros((n, n // 2), np.float32)
    od = np.zeros((n, n // 2), np.float32)
    ev[np.arange(0, n, 2), np.arange(n // 2)] = 1.0
    od[np.arange(1, n, 2), np.arange(n // 2)] = 1.0
    return jnp.asarray(ev), jnp.asarray(od)


def _ln(t):
    m = jnp.mean(t, axis=-1, keepdims=True)
    v = jnp.mean((t - m) ** 2, axis=-1, keepdims=True)
    return (t - m) / jnp.sqrt(v + EPS)


def _trunk_kernel(xt_ref, gam_ref, bet_ref,
                  pe1_ref, po1_ref, pe2_ref, po2_ref, pe3_ref, po3_ref,
                  w0_ref, b0_ref, w1_ref, b1_ref, w2_ref, b2_ref,
                  bd1_ref, bt1_ref, bd2_ref, bt2_ref, wct_ref, bc_ref,
                  wg_ref,
                  h_ref, gates_ref, a_ref, bden_ref, imp_ref):
    x = xt_ref[0]
    m = jnp.mean(x, axis=1, keepdims=True)
    v = jnp.mean((x - m) ** 2, axis=1, keepdims=True)
    std = jnp.sqrt(v + EPS)
    gam = gam_ref[...]
    bet = bet_ref[...]
    h = (x - m) / std * gam + bet

    hp = jax.lax.Precision.HIGHEST

    def _pool(t, pe_ref, po_ref):
        ev = jnp.dot(t, pe_ref[...], preferred_element_type=jnp.float32,
                     precision=hp)
        od = jnp.dot(t, po_ref[...], preferred_element_type=jnp.float32,
                     precision=hp)
        return (ev + od) * 0.5

    xs0 = h
    xs1 = _pool(xs0, pe1_ref, po1_ref)
    xs2 = _pool(xs1, pe2_ref, po2_ref)
    xs3 = _pool(xs2, pe3_ref, po3_ref)
    temb = xs3
    for xs_prev, w_r, b_r in ((xs2, w0_ref, b0_ref),
                              (xs1, w1_ref, b1_ref),
                              (xs0, w2_ref, b2_ref)):
        t = _ln(temb)
        up = jax.nn.gelu(jnp.dot(t.astype(jnp.bfloat16), w_r[...],
                                 preferred_element_type=jnp.float32) + b_r[...])
        temb = xs_prev + up

    for n in range(NBLOCK):
        res = h
        hh = _ln(h)
        hhb = hh.astype(jnp.bfloat16)
        t1 = jax.nn.gelu(jnp.dot(hhb, bd1_ref[n],
                                 preferred_element_type=jnp.float32) + bt1_ref[n])
        xt = jnp.dot(t1.astype(jnp.bfloat16), bd2_ref[n],
                     preferred_element_type=jnp.float32) + bt2_ref[n]
        xc = jnp.dot(wct_ref[n], hhb,
                     preferred_element_type=jnp.float32) + bc_ref[n]
        h = res + ALPHA * xt + (1.0 - ALPHA) * xc

    logits = jnp.dot(temb.astype(jnp.bfloat16), wg_ref[...],
                     preferred_element_type=jnp.float32)
    lane = jax.lax.broadcasted_iota(jnp.int32, logits.shape, 1)
    neg = jnp.float32(-1e30)
    big = jnp.int32(10 ** 9)
    lg = jnp.where(lane < E, logits, neg)
    m1 = jnp.max(lg, axis=1, keepdims=True)
    i1 = jnp.min(jnp.where(lg == m1, lane, big), axis=1, keepdims=True)
    lg2 = jnp.where(lane == i1, neg, lg)
    m2 = jnp.max(lg2, axis=1, keepdims=True)
    i2 = jnp.min(jnp.where(lg2 == m2, lane, big), axis=1, keepdims=True)
    w1g = jax.nn.sigmoid(m1 - m2)
    w2g = 1.0 - w1g
    row = jax.lax.broadcasted_iota(jnp.int32, logits.shape, 0)
    rmask = (row < C).astype(jnp.float32)
    gates = (jnp.where(lane == i1, w1g, 0.0)
             + jnp.where(lane == i2, w2g, 0.0)) * rmask
    gates8 = gates[:, :E]

    h_ref[0] = h
    gates_ref[0] = gates8
    imp_ref[0] = jnp.sum(gates8, axis=0, keepdims=True)
    a = std / (gam + EPS * EPS)
    a_ref[0] = a
    bden_ref[0] = m - bet * a


def _moe_kernel(hb_ref, g_ref, w1_ref, b1_ref, w2_ref, b2_ref,
                a_ref, bden_ref, imp_ref, out_ref, loss_ref, acc_ref):
    e = pl.program_id(0)
    t = pl.program_id(1)
    hblk = hb_ref[...]
    z = jnp.dot(hblk, w1_ref[0],
                preferred_element_type=jnp.float32) + b1_ref[0]
    hid = jax.nn.gelu(z.astype(jnp.bfloat16))
    y = jnp.dot(hid, w2_ref[0],
                preferred_element_type=jnp.float32) + b2_ref[0]
    g = g_ref[...]
    lane = jax.lax.broadcasted_iota(jnp.int32, g.shape, 1)
    gcol = jnp.sum(jnp.where(lane == e, g, 0.0), axis=1, keepdims=True)
    contrib = gcol * y
    sl = pl.dslice(t * BT, BT)

    @pl.when(e == 0)
    def _():
        acc_ref[sl, :] = contrib

    @pl.when((e > 0) & (e < E - 1))
    def _():
        acc_ref[sl, :] = acc_ref[sl, :] + contrib

    @pl.when(e == E - 1)
    def _():
        out_ref[...] = (acc_ref[sl, :] + contrib) * a_ref[...] + bden_ref[...]

    @pl.when((t == NT - 1) & (e == E - 1))
    def _():
        imp = jnp.sum(imp_ref[:, 0, :], axis=0, keepdims=True)
        mi = jnp.mean(imp, axis=1, keepdims=True)
        va = jnp.mean((imp - mi) ** 2, axis=1, keepdims=True)
        loss_ref[...] = va / (mi * mi + 1e-10)


def _run_trunk(x, gamma, beta, mdm_W0, mdm_b0, mdm_W1, mdm_b1, mdm_W2, mdm_b2,
               ddi0_Wt1, ddi0_bt1, ddi0_Wt2, ddi0_bt2, ddi0_Wc, ddi0_bc,
               ddi1_Wt1, ddi1_bt1, ddi1_Wt2, ddi1_bt2, ddi1_Wc, ddi1_bc,
               w_gate):
    f32 = jnp.float32
    xt = jnp.transpose(x, (0, 2, 1))
    xt = jnp.pad(xt, ((0, 0), (0, CP - C), (0, 0)))
    gam = jnp.pad(gamma, (0, CP - C), constant_values=1.0).reshape(CP, 1)
    bet = jnp.pad(beta, (0, CP - C)).reshape(CP, 1)
    pe1, po1 = _sel_mats(L)
    pe2, po2 = _sel_mats(L // 2)
    pe3, po3 = _sel_mats(L // 4)

    eye = jnp.eye(NPATCH, dtype=f32)
    bd1 = jnp.stack([jnp.kron(eye, ddi0_Wt1), jnp.kron(eye, ddi1_Wt1)])
    bd2 = jnp.stack([jnp.kron(eye, ddi0_Wt2), jnp.kron(eye, ddi1_Wt2)])
    bt1 = jnp.stack([jnp.tile(ddi0_bt1, NPATCH), jnp.tile(ddi1_bt1, NPATCH)])
    bt2 = jnp.stack([jnp.tile(ddi0_bt2, NPATCH), jnp.tile(ddi1_bt2, NPATCH)])
    bt1 = bt1.reshape(2, 1, 2 * L)
    bt2 = bt2.reshape(2, 1, L)
    wct = jnp.stack([
        jnp.pad(ddi0_Wc, ((0, CP - C), (0, CP - C))).T,
        jnp.pad(ddi1_Wc, ((0, CP - C), (0, CP - C))).T,
    ]).astype(jnp.bfloat16)
    bd1 = bd1.astype(jnp.bfloat16)
    bd2 = bd2.astype(jnp.bfloat16)
    bc = jnp.stack([jnp.pad(ddi0_bc, (0, CP - C)),
                    jnp.pad(ddi1_bc, (0, CP - C))]).reshape(2, CP, 1)
    wgp = jnp.pad(w_gate, ((0, 0), (0, 128 - E))).astype(jnp.bfloat16)
    b0r = mdm_b0.reshape(1, -1)
    b1r = mdm_b1.reshape(1, -1)
    b2r = mdm_b2.reshape(1, -1)

    cst = lambda *bs: pl.BlockSpec(bs, lambda b: (0,) * len(bs))
    h_out, gates, atok, btok, imp = pl.pallas_call(
        _trunk_kernel,
        grid=(B,),
        in_specs=[
            pl.BlockSpec((1, CP, L), lambda b: (b, 0, 0)),
            cst(CP, 1), cst(CP, 1),
            cst(L, L // 2), cst(L, L // 2),
            cst(L // 2, L // 4), cst(L // 2, L // 4),
            cst(L // 4, L // 8), cst(L // 4, L // 8),
            cst(64, 128), cst(1, 128),
            cst(128, 256), cst(1, 256),
            cst(256, 512), cst(1, 512),
            cst(2, L, 2 * L), cst(2, 1, 2 * L),
            cst(2, 2 * L, L), cst(2, 1, L),
            cst(2, CP, CP), cst(2, CP, 1),
            cst(L, 128),
        ],
        out_specs=[
            pl.BlockSpec((1, CP, L), lambda b: (b, 0, 0)),
            pl.BlockSpec((1, CP, E), lambda b: (b, 0, 0)),
            pl.BlockSpec((1, CP, 1), lambda b: (b, 0, 0)),
            pl.BlockSpec((1, CP, 1), lambda b: (b, 0, 0)),
            pl.BlockSpec((1, 1, E), lambda b: (b, 0, 0)),
        ],
        out_shape=[
            jax.ShapeDtypeStruct((B, CP, L), f32),
            jax.ShapeDtypeStruct((B, CP, E), f32),
            jax.ShapeDtypeStruct((B, CP, 1), f32),
            jax.ShapeDtypeStruct((B, CP, 1), f32),
            jax.ShapeDtypeStruct((B, 1, E), f32),
        ],
    )(xt, gam, bet, pe1, po1, pe2, po2, pe3, po3,
      mdm_W0.astype(jnp.bfloat16), b0r,
      mdm_W1.astype(jnp.bfloat16), b1r, mdm_W2.astype(jnp.bfloat16), b2r,
      bd1, bt1, bd2, bt2, wct, bc, wgp)
    return h_out, gates, atok, btok, imp


def _run_moe(h_out, gates, atok, btok, imp, expW1, expb1, expW2, expb2):
    f32 = jnp.float32
    hb = h_out.reshape(NTOK, L).astype(jnp.bfloat16)
    gtok = gates.reshape(NTOK, E)
    a_tok = atok.reshape(NTOK, 1)
    b_tok = btok.reshape(NTOK, 1)
    w1b = expW1.astype(jnp.bfloat16)
    w2b = jnp.pad(expW2, ((0, 0), (0, 0), (0, PP - PRED))).astype(jnp.bfloat16)
    b1e = expb1.reshape(E, 1, FF)
    b2e = jnp.pad(expb2, ((0, 0), (0, PP - PRED))).reshape(E, 1, PP)

    out_tok, loss = pl.pallas_call(
        _moe_kernel,
        grid=(E, NT),
        in_specs=[
            pl.BlockSpec((BT, L), lambda e, t: (t, 0)),
            pl.BlockSpec((BT, E), lambda e, t: (t, 0)),
            pl.BlockSpec((1, L, FF), lambda e, t: (e, 0, 0)),
            pl.BlockSpec((1, 1, FF), lambda e, t: (e, 0, 0)),
            pl.BlockSpec((1, FF, PP), lambda e, t: (e, 0, 0)),
            pl.BlockSpec((1, 1, PP), lambda e, t: (e, 0, 0)),
            pl.BlockSpec((BT, 1), lambda e, t: (t, 0)),
            pl.BlockSpec((BT, 1), lambda e, t: (t, 0)),
            pl.BlockSpec((B, 1, E), lambda e, t: (0, 0, 0)),
        ],
        out_specs=[
            pl.BlockSpec((BT, PP), lambda e, t: (t, 0)),
            pl.BlockSpec((1, 1), lambda e, t: (0, 0)),
        ],
        out_shape=[
            jax.ShapeDtypeStruct((NTOK, PP), f32),
            jax.ShapeDtypeStruct((1, 1), f32),
        ],
        scratch_shapes=[pltpu.VMEM((NTOK, PP), f32)],
    )(hb, gtok, w1b, b1e, w2b, b2e, a_tok, b_tok, imp)

    out = out_tok[:, :PRED].reshape(B, CP, PRED)[:, :C, :]
    out = jnp.transpose(out, (0, 2, 1))
    return out, loss[0, 0]


def kernel(x, gamma, beta, mdm_W0, mdm_b0, mdm_W1, mdm_b1, mdm_W2, mdm_b2,
           ddi0_Wt1, ddi0_bt1, ddi0_Wt2, ddi0_bt2, ddi0_Wc, ddi0_bc,
           ddi1_Wt1, ddi1_bt1, ddi1_Wt2, ddi1_bt2, ddi1_Wc, ddi1_bc,
           w_gate, expW1, expb1, expW2, expb2):
    trunk = _run_trunk(x, gamma, beta, mdm_W0, mdm_b0, mdm_W1, mdm_b1,
                       mdm_W2, mdm_b2,
                       ddi0_Wt1, ddi0_bt1, ddi0_Wt2, ddi0_bt2, ddi0_Wc,
                       ddi0_bc,
                       ddi1_Wt1, ddi1_bt1, ddi1_Wt2, ddi1_bt2, ddi1_Wc,
                       ddi1_bc, w_gate)
    return _run_moe(*trunk, expW1, expb1, expW2, expb2)


# speedup vs baseline: 1.0201x; 1.0201x over previous
import jax
import jax.numpy as jnp
import numpy as np
from jax.experimental import pallas as pl
from jax.experimental.pallas import tpu as pltpu

B, L, C = 8, 512, 321
PRED, PATCH, K, CC, ALPHA = 96, 16, 3, 2, 0.5
E, TOPK, FF = 8, 2, 2048
NBLOCK = 2
EPS = 1e-5

CP = 384
NTOK = B * CP
BT = 1024
NT = NTOK // BT
PP = 128
NPATCH = L // PATCH


def _sel_mats(n):
    ev = np.zeros((n, n // 2), np.float32)
    od = np.zeros((n, n // 2), np.float32)
    ev[np.arange(0, n, 2), np.arange(n // 2)] = 1.0
    od[np.arange(1, n, 2), np.arange(n // 2)] = 1.0
    return jnp.asarray(ev), jnp.asarray(od)


def _ln(t):
    m = jnp.mean(t, axis=-1, keepdims=True)
    v = jnp.mean((t - m) ** 2, axis=-1, keepdims=True)
    return (t - m) / jnp.sqrt(v + EPS)


def _trunk_kernel(xt_ref, gam_ref, bet_ref,
                  pe1_ref, po1_ref, pe2_ref, po2_ref, pe3_ref, po3_ref,
                  w0_ref, b0_ref, w1_ref, b1_ref, w2_ref, b2_ref,
                  bd1_ref, bt1_ref, bd2_ref, bt2_ref, wct_ref, bc_ref,
                  wg_ref,
                  h_ref, gates_ref, a_ref, bden_ref, imp_ref):
    x = xt_ref[0]
    m = jnp.mean(x, axis=1, keepdims=True)
    v = jnp.mean((x - m) ** 2, axis=1, keepdims=True)
    std = jnp.sqrt(v + EPS)
    gam = gam_ref[...]
    bet = bet_ref[...]
    h = (x - m) / std * gam + bet

    hp = jax.lax.Precision.HIGHEST

    def _pool(t, pe_ref, po_ref):
        ev = jnp.dot(t, pe_ref[...], preferred_element_type=jnp.float32,
                     precision=hp)
        od = jnp.dot(t, po_ref[...], preferred_element_type=jnp.float32,
                     precision=hp)
        return (ev + od) * 0.5

    xs0 = h
    xs1 = _pool(xs0, pe1_ref, po1_ref)
    xs2 = _pool(xs1, pe2_ref, po2_ref)
    xs3 = _pool(xs2, pe3_ref, po3_ref)
    temb = xs3
    for xs_prev, w_r, b_r in ((xs2, w0_ref, b0_ref),
                              (xs1, w1_ref, b1_ref),
                              (xs0, w2_ref, b2_ref)):
        t = _ln(temb)
        up = jax.nn.gelu(jnp.dot(t.astype(jnp.bfloat16), w_r[...],
                                 preferred_element_type=jnp.float32) + b_r[...])
        temb = xs_prev + up

    for n in range(NBLOCK):
        res = h
        hh = _ln(h)
        hhb = hh.astype(jnp.bfloat16)
        t1 = jax.nn.gelu(jnp.dot(hhb, bd1_ref[n],
                                 preferred_element_type=jnp.float32) + bt1_ref[n])
        xt = jnp.dot(t1.astype(jnp.bfloat16), bd2_ref[n],
                     preferred_element_type=jnp.float32) + bt2_ref[n]
        xc = jnp.dot(wct_ref[n], hhb,
                     preferred_element_type=jnp.float32) + bc_ref[n]
        h = res + ALPHA * xt + (1.0 - ALPHA) * xc

    logits = jnp.dot(temb.astype(jnp.bfloat16), wg_ref[...],
                     preferred_element_type=jnp.float32)
    lane = jax.lax.broadcasted_iota(jnp.int32, logits.shape, 1)
    neg = jnp.float32(-1e30)
    big = jnp.int32(10 ** 9)
    lg = jnp.where(lane < E, logits, neg)
    m1 = jnp.max(lg, axis=1, keepdims=True)
    i1 = jnp.min(jnp.where(lg == m1, lane, big), axis=1, keepdims=True)
    lg2 = jnp.where(lane == i1, neg, lg)
    m2 = jnp.max(lg2, axis=1, keepdims=True)
    i2 = jnp.min(jnp.where(lg2 == m2, lane, big), axis=1, keepdims=True)
    w1g = jax.nn.sigmoid(m1 - m2)
    w2g = 1.0 - w1g
    row = jax.lax.broadcasted_iota(jnp.int32, logits.shape, 0)
    rmask = (row < C).astype(jnp.float32)
    gates = (jnp.where(lane == i1, w1g, 0.0)
             + jnp.where(lane == i2, w2g, 0.0)) * rmask
    gates8 = gates[:, :E]

    h_ref[0] = h.astype(jnp.bfloat16)
    gates_ref[0] = gates8
    imp_ref[0] = jnp.sum(gates8, axis=0, keepdims=True)
    a = std / (gam + EPS * EPS)
    a_ref[0] = a
    bden_ref[0] = m - bet * a


def _moe_kernel(hb_ref, g_ref, w1_ref, b1_ref, w2_ref, b2_ref,
                a_ref, bden_ref, imp_ref, out_ref, loss_ref, acc_ref):
    e = pl.program_id(0)
    t = pl.program_id(1)
    hblk = hb_ref[...]
    z = jnp.dot(hblk, w1_ref[0],
                preferred_element_type=jnp.float32) + b1_ref[0]
    hid = jax.nn.gelu(z.astype(jnp.bfloat16))
    y = jnp.dot(hid, w2_ref[0],
                preferred_element_type=jnp.float32) + b2_ref[0]
    g = g_ref[...]
    lane = jax.lax.broadcasted_iota(jnp.int32, g.shape, 1)
    gcol = jnp.sum(jnp.where(lane == e, g, 0.0), axis=1, keepdims=True)
    contrib = gcol * y
    sl = pl.dslice(t * BT, BT)

    @pl.when(e == 0)
    def _():
        acc_ref[sl, :] = contrib

    @pl.when((e > 0) & (e < E - 1))
    def _():
        acc_ref[sl, :] = acc_ref[sl, :] + contrib

    @pl.when(e == E - 1)
    def _():
        out_ref[...] = (acc_ref[sl, :] + contrib) * a_ref[...] + bden_ref[...]

    @pl.when((t == NT - 1) & (e == E - 1))
    def _():
        imp = jnp.sum(imp_ref[:, 0, :], axis=0, keepdims=True)
        mi = jnp.mean(imp, axis=1, keepdims=True)
        va = jnp.mean((imp - mi) ** 2, axis=1, keepdims=True)
        loss_ref[...] = va / (mi * mi + 1e-10)


def _run_trunk(x, gamma, beta, mdm_W0, mdm_b0, mdm_W1, mdm_b1, mdm_W2, mdm_b2,
               ddi0_Wt1, ddi0_bt1, ddi0_Wt2, ddi0_bt2, ddi0_Wc, ddi0_bc,
               ddi1_Wt1, ddi1_bt1, ddi1_Wt2, ddi1_bt2, ddi1_Wc, ddi1_bc,
               w_gate):
    f32 = jnp.float32
    xt = jnp.transpose(x, (0, 2, 1))
    xt = jnp.pad(xt, ((0, 0), (0, CP - C), (0, 0)))
    gam = jnp.pad(gamma, (0, CP - C), constant_values=1.0).reshape(CP, 1)
    bet = jnp.pad(beta, (0, CP - C)).reshape(CP, 1)
    pe1, po1 = _sel_mats(L)
    pe2, po2 = _sel_mats(L // 2)
    pe3, po3 = _sel_mats(L // 4)

    eye = jnp.eye(NPATCH, dtype=f32)
    bd1 = jnp.stack([jnp.kron(eye, ddi0_Wt1), jnp.kron(eye, ddi1_Wt1)])
    bd2 = jnp.stack([jnp.kron(eye, ddi0_Wt2), jnp.kron(eye, ddi1_Wt2)])
    bt1 = jnp.stack([jnp.tile(ddi0_bt1, NPATCH), jnp.tile(ddi1_bt1, NPATCH)])
    bt2 = jnp.stack([jnp.tile(ddi0_bt2, NPATCH), jnp.tile(ddi1_bt2, NPATCH)])
    bt1 = bt1.reshape(2, 1, 2 * L)
    bt2 = bt2.reshape(2, 1, L)
    wct = jnp.stack([
        jnp.pad(ddi0_Wc, ((0, CP - C), (0, CP - C))).T,
        jnp.pad(ddi1_Wc, ((0, CP - C), (0, CP - C))).T,
    ]).astype(jnp.bfloat16)
    bd1 = bd1.astype(jnp.bfloat16)
    bd2 = bd2.astype(jnp.bfloat16)
    bc = jnp.stack([jnp.pad(ddi0_bc, (0, CP - C)),
                    jnp.pad(ddi1_bc, (0, CP - C))]).reshape(2, CP, 1)
    wgp = jnp.pad(w_gate, ((0, 0), (0, 128 - E))).astype(jnp.bfloat16)
    b0r = mdm_b0.reshape(1, -1)
    b1r = mdm_b1.reshape(1, -1)
    b2r = mdm_b2.reshape(1, -1)

    cst = lambda *bs: pl.BlockSpec(bs, lambda b: (0,) * len(bs))
    h_out, gates, atok, btok, imp = pl.pallas_call(
        _trunk_kernel,
        grid=(B,),
        in_specs=[
            pl.BlockSpec((1, CP, L), lambda b: (b, 0, 0)),
            cst(CP, 1), cst(CP, 1),
            cst(L, L // 2), cst(L, L // 2),
            cst(L // 2, L // 4), cst(L // 2, L // 4),
            cst(L // 4, L // 8), cst(L // 4, L // 8),
            cst(64, 128), cst(1, 128),
            cst(128, 256), cst(1, 256),
            cst(256, 512), cst(1, 512),
            cst(2, L, 2 * L), cst(2, 1, 2 * L),
            cst(2, 2 * L, L), cst(2, 1, L),
            cst(2, CP, CP), cst(2, CP, 1),
            cst(L, 128),
        ],
        out_specs=[
            pl.BlockSpec((1, CP, L), lambda b: (b, 0, 0)),
            pl.BlockSpec((1, CP, E), lambda b: (b, 0, 0)),
            pl.BlockSpec((1, CP, 1), lambda b: (b, 0, 0)),
            pl.BlockSpec((1, CP, 1), lambda b: (b, 0, 0)),
            pl.BlockSpec((1, 1, E), lambda b: (b, 0, 0)),
        ],
        out_shape=[
            jax.ShapeDtypeStruct((B, CP, L), jnp.bfloat16),
            jax.ShapeDtypeStruct((B, CP, E), f32),
            jax.ShapeDtypeStruct((B, CP, 1), f32),
            jax.ShapeDtypeStruct((B, CP, 1), f32),
            jax.ShapeDtypeStruct((B, 1, E), f32),
        ],
    )(xt, gam, bet, pe1, po1, pe2, po2, pe3, po3,
      mdm_W0.astype(jnp.bfloat16), b0r,
      mdm_W1.astype(jnp.bfloat16), b1r, mdm_W2.astype(jnp.bfloat16), b2r,
      bd1, bt1, bd2, bt2, wct, bc, wgp)
    return h_out, gates, atok, btok, imp


def _run_moe(h_out, gates, atok, btok, imp, expW1, expb1, expW2, expb2):
    f32 = jnp.float32
    hb = h_out.reshape(NTOK, L)
    gtok = gates.reshape(NTOK, E)
    a_tok = atok.reshape(NTOK, 1)
    b_tok = btok.reshape(NTOK, 1)
    w1b = expW1.astype(jnp.bfloat16)
    w2b = jnp.pad(expW2, ((0, 0), (0, 0), (0, PP - PRED))).astype(jnp.bfloat16)
    b1e = expb1.reshape(E, 1, FF)
    b2e = jnp.pad(expb2, ((0, 0), (0, PP - PRED))).reshape(E, 1, PP)

    out_tok, loss = pl.pallas_call(
        _moe_kernel,
        grid=(E, NT),
        in_specs=[
            pl.BlockSpec((BT, L), lambda e, t: (t, 0)),
            pl.BlockSpec((BT, E), lambda e, t: (t, 0)),
            pl.BlockSpec((1, L, FF), lambda e, t: (e, 0, 0)),
            pl.BlockSpec((1, 1, FF), lambda e, t: (e, 0, 0)),
            pl.BlockSpec((1, FF, PP), lambda e, t: (e, 0, 0)),
            pl.BlockSpec((1, 1, PP), lambda e, t: (e, 0, 0)),
            pl.BlockSpec((BT, 1), lambda e, t: (t, 0)),
            pl.BlockSpec((BT, 1), lambda e, t: (t, 0)),
            pl.BlockSpec((B, 1, E), lambda e, t: (0, 0, 0)),
        ],
        out_specs=[
            pl.BlockSpec((BT, PP), lambda e, t: (t, 0)),
            pl.BlockSpec((1, 1), lambda e, t: (0, 0)),
        ],
        out_shape=[
            jax.ShapeDtypeStruct((NTOK, PP), f32),
            jax.ShapeDtypeStruct((1, 1), f32),
        ],
        scratch_shapes=[pltpu.VMEM((NTOK, PP), f32)],
    )(hb, gtok, w1b, b1e, w2b, b2e, a_tok, b_tok, imp)

    out = out_tok[:, :PRED].reshape(B, CP, PRED)[:, :C, :]
    out = jnp.transpose(out, (0, 2, 1))
    return out, loss[0, 0]


def kernel(x, gamma, beta, mdm_W0, mdm_b0, mdm_W1, mdm_b1, mdm_W2, mdm_b2,
           ddi0_Wt1, ddi0_bt1, ddi0_Wt2, ddi0_bt2, ddi0_Wc, ddi0_bc,
           ddi1_Wt1, ddi1_bt1, ddi1_Wt2, ddi1_bt2, ddi1_Wc, ddi1_bc,
           w_gate, expW1, expb1, expW2, expb2):
    trunk = _run_trunk(x, gamma, beta, mdm_W0, mdm_b0, mdm_W1, mdm_b1,
                       mdm_W2, mdm_b2,
                       ddi0_Wt1, ddi0_bt1, ddi0_Wt2, ddi0_bt2, ddi0_Wc,
                       ddi0_bc,
                       ddi1_Wt1, ddi1_bt1, ddi1_Wt2, ddi1_bt2, ddi1_Wc,
                       ddi1_bc, w_gate)
    return _run_moe(*trunk, expW1, expb1, expW2, expb2)
